# baseline (speedup 1.0000x reference)
import functools

import jax
import jax.numpy as jnp
from jax import lax
from jax.experimental import pallas as pl
from jax.experimental.pallas import tpu as pltpu
from jax.experimental.pallas import tpu_sc as plsc

NA = 50000
NT = 50000
E = 800000
H = 64

RW = 32
CW = 16
NP = 50008
ACC_R = 50048
TPT = ACC_R // 16
E_PAD = 802816
EROWS = E_PAD // 128
TILE_EROWS = EROWS // 16
EH_TILE_EROWS = EROWS // 32

FKB = 2
FCHUNK = FKB * 128
NFCH = TILE_EROWS // FKB

CKB = 8
CCHUNK = CKB * 128
NCCH = TILE_EROWS // CKB

EKB = 2
ECHUNK = EKB * 128
NECH = EH_TILE_EROWS // EKB

ENC_R = 2000
POST_R = 2000

_SC_PARAMS = pltpu.CompilerParams(use_tc_tiling_on_sc=False,
                                  needs_layout_passes=False)


def _enc_body(x_ref, w_ref, b_ref, g_ref, be_ref, h_ref, gtab_ref):
    x = x_ref[...]
    y = jnp.dot(x, w_ref[...].T, preferred_element_type=jnp.float32)
    y = y + b_ref[...]
    m = jnp.mean(y, axis=-1, keepdims=True)
    v = jnp.mean((y - m) ** 2, axis=-1, keepdims=True)
    y = (y - m) * lax.rsqrt(v + 1e-5) * g_ref[...] + be_ref[...]
    h = jnp.maximum(y, 0.0)
    h_ref[...] = h
    gtab_ref[...] = jnp.stack([h[:, :RW], h[:, RW:]], axis=0)


def _encode(x, w, b, g, be, n):
    return pl.pallas_call(
        _enc_body,
        grid=(n // ENC_R,),
        in_specs=[
            pl.BlockSpec((ENC_R, 128), lambda i: (i, 0)),
            pl.BlockSpec((H, 128), lambda i: (0, 0)),
            pl.BlockSpec((H,), lambda i: (0,)),
            pl.BlockSpec((H,), lambda i: (0,)),
            pl.BlockSpec((H,), lambda i: (0,)),
        ],
        out_specs=[
            pl.BlockSpec((ENC_R, H), lambda i: (i, 0)),
            pl.BlockSpec((2, ENC_R, RW), lambda i: (0, i, 0)),
        ],
        out_shape=[
            jax.ShapeDtypeStruct((n, H), jnp.float32),
            jax.ShapeDtypeStruct((2, n, RW), jnp.float32),
        ],
    )(x, w, b, g, be)


def _ring_driver(n, S, G):
    for k in range(3):
        G(k, k, first=True)

    def body(jj, _):
        k0 = 3 * jj
        for u in range(3):
            S(k0 + u, u)
            G(k0 + u + 3, u)
        return _

    nloop = (n - 4) // 3
    lax.fori_loop(0, nloop, body, None)
    for k in range(3 * nloop, n):
        S(k, k % 3)
        kg = k + 3
        if kg < n:
            G(kg, kg % 3)


def _feat_phase(c, s, table, gpar, out, acc, cidx, rows, gsems, ssems,
                zf_hbm, comb):
    pltpu.sync_copy(zf_hbm, acc.at[pl.ds(s * TPT, TPT)])
    plsc.subcore_barrier()
    base = 2 * s * TILE_EROWS
    cnp = c * NP

    def G(k, b, first=False):
        if not first:
            for j in range(FKB):
                pltpu.make_async_copy(rows[b].at[pl.ds(j * 128, 128)],
                                      acc.at[pl.ds(0, 128)], ssems[b]).wait()
        r0 = base + k * (2 * FKB)
        pltpu.sync_copy(comb.at[pl.ds(r0, 2 * FKB)], cidx[b])
        for j in range(FKB):
            row = 2 * j + gpar
            for q in range(8):
                sl = cidx[b][row, pl.ds(q * 16, 16)]
                cidx[b][row, pl.ds(q * 16, 16)] = sl + cnp
            pltpu.async_copy(table.at[cidx[b].at[row]],
                             rows[b].at[pl.ds(j * 128, 128)], gsems[b])

    def S(k, b):
        for j in range(FKB):
            pltpu.make_async_copy(table.at[pl.ds(0, 128)],
                                  rows[b].at[pl.ds(j * 128, 128)],
                                  gsems[b]).wait()
        for j in range(FKB):
            pltpu.async_copy(rows[b].at[pl.ds(j * 128, 128)],
                             acc.at[cidx[b].at[2 * j + 1 - gpar]], ssems[b],
                             add=True)

    _ring_driver(NFCH, S, G)
    for b in range(3):
        for j in range(FKB):
            pltpu.make_async_copy(rows[b].at[pl.ds(j * 128, 128)],
                                  acc.at[pl.ds(0, 128)], ssems[b]).wait()
    plsc.subcore_barrier()
    pltpu.sync_copy(acc.at[pl.ds(s * TPT, TPT)],
                    out.at[c, pl.ds(s * TPT, TPT)])
    plsc.subcore_barrier()


def _cnt_phase(c, s, out, acc, cidx, ones_v, ssems, zf_hbm, comb):
    pltpu.sync_copy(zf_hbm, acc.at[pl.ds(s * TPT, TPT)])
    plsc.subcore_barrier()
    base = 2 * s * TILE_EROWS

    def G(k, b, first=False):
        if not first:
            for j in range(FKB):
                pltpu.make_async_copy(ones_v, acc.at[pl.ds(0, 128)],
                                      ssems[b]).wait()
        r0 = base + k * (2 * FKB)
        pltpu.sync_copy(comb.at[pl.ds(r0, 2 * FKB)], cidx[b])

    def S(k, b):
        @pl.when(c == 0)
        def _():
            for j in range(FKB):
                pltpu.async_copy(ones_v, acc.at[cidx[b].at[2 * j + 1]],
                                 ssems[b], add=True)

        @pl.when(c == 1)
        def _():
            for j in range(FKB):
                pltpu.async_copy(ones_v, acc.at[cidx[b].at[2 * j]],
                                 ssems[b], add=True)

    _ring_driver(NFCH, S, G)
    for b in range(3):
        for j in range(FKB):
            pltpu.make_async_copy(ones_v, acc.at[pl.ds(0, 128)],
                                  ssems[b]).wait()
    plsc.subcore_barrier()
    pltpu.sync_copy(acc.at[pl.ds(s * TPT, TPT)],
                    out.at[c, pl.ds(s * TPT, TPT)])


def _feat_body(g_a, g_t, comb, ones_hbm, zf_hbm,
               s_t, s_a, cnt, acc, cidx0, cidx1, cidx2, rows0, rows1, rows2,
               ones_v, gs0, gs1, gs2, ss0, ss1, ss2):
    c = lax.axis_index("c")
    s = lax.axis_index("s")
    cidx = (cidx0, cidx1, cidx2)
    rows = (rows0, rows1, rows2)
    gsems = (gs0, gs1, gs2)
    ssems = (ss0, ss1, ss2)
    pltpu.sync_copy(ones_hbm, ones_v)
    _feat_phase(c, s, g_a, 0, s_t, acc, cidx, rows, gsems, ssems,
                zf_hbm, comb)
    _feat_phase(c, s, g_t, 1, s_a, acc, cidx, rows, gsems, ssems,
                zf_hbm, comb)
    _cnt_phase(c, s, cnt, acc, cidx, ones_v, ssems, zf_hbm, comb)


def _feat(g_a, g_t, comb, ones_hbm, zf_hbm):
    mesh = plsc.VectorSubcoreMesh(core_axis_name="c", subcore_axis_name="s")
    f = functools.partial(
        pl.kernel,
        out_type=[
            jax.ShapeDtypeStruct((2, ACC_R, RW), jnp.float32),
            jax.ShapeDtypeStruct((2, ACC_R, RW), jnp.float32),
            jax.ShapeDtypeStruct((2, ACC_R, RW), jnp.float32),
        ],
        mesh=mesh,
        compiler_params=_SC_PARAMS,
        scratch_types=[
            pltpu.VMEM_SHARED((ACC_R, RW), jnp.float32),
            pltpu.VMEM((2 * FKB, 128), jnp.int32),
            pltpu.VMEM((2 * FKB, 128), jnp.int32),
            pltpu.VMEM((2 * FKB, 128), jnp.int32),
            pltpu.VMEM((FCHUNK, RW), jnp.float32),
            pltpu.VMEM((FCHUNK, RW), jnp.float32),
            pltpu.VMEM((FCHUNK, RW), jnp.float32),
            pltpu.VMEM((128, RW), jnp.float32),
            pltpu.SemaphoreType.DMA,
            pltpu.SemaphoreType.DMA,
            pltpu.SemaphoreType.DMA,
            pltpu.SemaphoreType.DMA,
            pltpu.SemaphoreType.DMA,
            pltpu.SemaphoreType.DMA,
        ],
    )(_feat_body)
    return f(g_a, g_t, comb, ones_hbm, zf_hbm)


def _post_body(st_ref, sa_ref, cnt_ref, ha_ref, ht_ref, wlst_ref, blst_ref,
               wrst_ref, wlts_ref, blts_ref, wrts_ref, wh1_ref, bh1_ref,
               a_ref, t_ref):
    st = st_ref[...]
    sa = sa_ref[...]
    cnt = cnt_ref[...]
    cnt_t = jnp.maximum(cnt[0, :, 0:1], 1.0)
    cnt_a = jnp.maximum(cnt[1, :, 0:1], 1.0)
    agg_t = jnp.concatenate([st[0], st[1]], axis=1) / cnt_t
    agg_a = jnp.concatenate([sa[0], sa[1]], axis=1) / cnt_a
    new_t = (jnp.dot(agg_t, wlst_ref[...].T, preferred_element_type=jnp.float32)
             + blst_ref[...]
             + jnp.dot(ht_ref[...], wrst_ref[...].T, preferred_element_type=jnp.float32))
    new_a = (jnp.dot(agg_a, wlts_ref[...].T, preferred_element_type=jnp.float32)
             + blts_ref[...]
             + jnp.dot(ha_ref[...], wrts_ref[...].T, preferred_element_type=jnp.float32))
    wh1 = wh1_ref[...]
    a_ref[...] = (jnp.dot(new_a, wh1[:, :H].T, preferred_element_type=jnp.float32)
                  + bh1_ref[...])
    t_ref[...] = jnp.dot(new_t, wh1[:, H:].T, preferred_element_type=jnp.float32)


def _post(s_t, s_a, cnt, h_a, h_t, wlst, blst, wrst, wlts, blts, wrts,
          wh1, bh1):
    return pl.pallas_call(
        _post_body,
        grid=(NA // POST_R,),
        in_specs=[
            pl.BlockSpec((2, POST_R, RW), lambda i: (0, i, 0)),
            pl.BlockSpec((2, POST_R, RW), lambda i: (0, i, 0)),
            pl.BlockSpec((2, POST_R, RW), lambda i: (0, i, 0)),
            pl.BlockSpec((POST_R, H), lambda i: (i, 0)),
            pl.BlockSpec((POST_R, H), lambda i: (i, 0)),
            pl.BlockSpec((H, H), lambda i: (0, 0)),
            pl.BlockSpec((H,), lambda i: (0,)),
            pl.BlockSpec((H, H), lambda i: (0, 0)),
            pl.BlockSpec((H, H), lambda i: (0, 0)),
            pl.BlockSpec((H,), lambda i: (0,)),
            pl.BlockSpec((H, H), lambda i: (0, 0)),
            pl.BlockSpec((H, 2 * H), lambda i: (0, 0)),
            pl.BlockSpec((H,), lambda i: (0,)),
        ],
        out_specs=[
            pl.BlockSpec((POST_R, H), lambda i: (i, 0)),
            pl.BlockSpec((POST_R, H), lambda i: (i, 0)),
        ],
        out_shape=[
            jax.ShapeDtypeStruct((ACC_R, H), jnp.float32),
            jax.ShapeDtypeStruct((ACC_R, H), jnp.float32),
        ],
    )(s_t, s_a, cnt, h_a, h_t, wlst, blst, wrst, wlts, blts, wrts, wh1, bh1)


def _edge_body(a_tab, t_tab, comb, w2_hbm, b2_hbm, logits,
               cidx0, cidx1, cidx2, ar0, ar1, ar2, tr0, tr1, tr2,
               out0, out1, out2, w2_v, b2_v, tile,
               sg0, sg1, sg2, sw0, sw1, sw2):
    c = lax.axis_index("c")
    s = lax.axis_index("s")
    wid = s * 2 + c
    base = 2 * wid * EH_TILE_EROWS
    obase = wid * (EH_TILE_EROWS * 128)
    cidx = (cidx0, cidx1, cidx2)
    ar = (ar0, ar1, ar2)
    tr = (tr0, tr1, tr2)
    outs = (out0, out1, out2)
    sg = (sg0, sg1, sg2)
    sw = (sw0, sw1, sw2)
    pltpu.sync_copy(w2_hbm, w2_v)
    pltpu.sync_copy(b2_hbm, b2_v)
    for b in range(3):
        pltpu.async_copy(outs[b], logits.at[pl.ds(E, ECHUNK)], sw[b])

    w2v = [w2_v[pl.ds(q * 16, 16)] for q in range(4)]
    b2vec = b2_v[...]
    iot = lax.iota(jnp.int32, 16)

    def G(k, b, first=False):
        r0 = base + k * (2 * EKB)
        pltpu.sync_copy(comb.at[pl.ds(r0, 2 * EKB)], cidx[b])
        for j in range(EKB):
            pltpu.async_copy(a_tab.at[cidx[b].at[2 * j]],
                             ar[b].at[pl.ds(j * 128, 128)], sg[b])
            pltpu.async_copy(t_tab.at[cidx[b].at[2 * j + 1]],
                             tr[b].at[pl.ds(j * 128, 128)], sg[b])

    def S(k, b):
        for j in range(EKB):
            pltpu.make_async_copy(a_tab.at[pl.ds(0, 128)],
                                  ar[b].at[pl.ds(j * 128, 128)], sg[b]).wait()
            pltpu.make_async_copy(t_tab.at[pl.ds(0, 128)],
                                  tr[b].at[pl.ds(j * 128, 128)], sg[b]).wait()
        out = outs[b]
        arows = ar[b]
        trows = tr[b]
        pltpu.make_async_copy(out, logits.at[pl.ds(E, ECHUNK)], sw[b]).wait()

        def group(g, _):
            e0 = g * 16
            for u in range(16):
                e = e0 + u
                t = None
                for q in range(4):
                    av = arows[e, pl.ds(q * 16, 16)]
                    tv = trows[e, pl.ds(q * 16, 16)]
                    r = jnp.maximum(av + tv, 0.0) * w2v[q]
                    t = r if t is None else t + r
                tile[u, :] = t
            res = b2vec
            for q in range(16):
                col = plsc.load_gather(tile, [iot, jnp.full((16,), q, jnp.int32)])
                res = res + col
            out[pl.ds(e0, 16)] = res
            return _

        lax.fori_loop(0, ECHUNK // 16, group, None)
        pltpu.async_copy(out, logits.at[pl.ds(obase + k * ECHUNK, ECHUNK)],
                         sw[b])

    _ring_driver(NECH, S, G)
    for b in range(3):
        pltpu.make_async_copy(outs[b], logits.at[pl.ds(E, ECHUNK)],
                              sw[b]).wait()


def _edge_head(a_tab, t_tab, comb, w2b, b2b):
    mesh = plsc.VectorSubcoreMesh(core_axis_name="c", subcore_axis_name="s")
    f = functools.partial(
        pl.kernel,
        out_type=jax.ShapeDtypeStruct((E_PAD,), jnp.float32),
        mesh=mesh,
        compiler_params=_SC_PARAMS,
        scratch_types=[
            pltpu.VMEM((2 * EKB, 128), jnp.int32),
            pltpu.VMEM((2 * EKB, 128), jnp.int32),
            pltpu.VMEM((2 * EKB, 128), jnp.int32),
            pltpu.VMEM((ECHUNK, H), jnp.float32),
            pltpu.VMEM((ECHUNK, H), jnp.float32),
            pltpu.VMEM((ECHUNK, H), jnp.float32),
            pltpu.VMEM((ECHUNK, H), jnp.float32),
            pltpu.VMEM((ECHUNK, H), jnp.float32),
            pltpu.VMEM((ECHUNK, H), jnp.float32),
            pltpu.VMEM((ECHUNK,), jnp.float32),
            pltpu.VMEM((ECHUNK,), jnp.float32),
            pltpu.VMEM((ECHUNK,), jnp.float32),
            pltpu.VMEM((H,), jnp.float32),
            pltpu.VMEM((16,), jnp.float32),
            pltpu.VMEM((16, 16), jnp.float32),
            pltpu.SemaphoreType.DMA,
            pltpu.SemaphoreType.DMA,
            pltpu.SemaphoreType.DMA,
            pltpu.SemaphoreType.DMA,
            pltpu.SemaphoreType.DMA,
            pltpu.SemaphoreType.DMA,
        ],
    )(_edge_body)
    return f(a_tab, t_tab, comb, w2b, b2b)


def kernel(x_agent, x_task, W_enc_a, b_enc_a, g_ln_a, be_ln_a, W_enc_t,
           b_enc_t, g_ln_t, be_ln_t, Wl_st, bl_st, Wr_st, Wl_ts, bl_ts,
           Wr_ts, W_h1, b_h1, W_h2, b_h2, edge_index):
    h_a, g_a = _encode(x_agent, W_enc_a, b_enc_a, g_ln_a, be_ln_a, NA)
    h_t, g_t = _encode(x_task, W_enc_t, b_enc_t, g_ln_t, be_ln_t, NT)
    zpad = jnp.zeros((2, NP - NA, RW), jnp.float32)
    g_a_flat = jnp.concatenate([g_a, zpad], axis=1).reshape(2 * NP, RW)
    g_t_flat = jnp.concatenate([g_t, zpad], axis=1).reshape(2 * NP, RW)

    src = edge_index[0]
    dst = edge_index[1]
    pad = E_PAD - E
    srcp = jnp.concatenate([src, jnp.full((pad,), NA, jnp.int32)])
    dstp = jnp.concatenate([dst, jnp.full((pad,), NA, jnp.int32)])
    comb = jnp.stack([srcp.reshape(EROWS, 128),
                      dstp.reshape(EROWS, 128)], axis=1)
    comb = comb.reshape(2 * EROWS, 128)

    ones_hbm = jnp.ones((128, RW), jnp.float32)
    zf_hbm = jnp.zeros((TPT, RW), jnp.float32)

    s_t, s_a, cnt = _feat(g_a_flat, g_t_flat, comb, ones_hbm, zf_hbm)

    a_tab, t_tab = _post(s_t, s_a, cnt, h_a, h_t, Wl_st, bl_st, Wr_st,
                         Wl_ts, bl_ts, Wr_ts, W_h1, b_h1)

    w2b = W_h2[0]
    b2b = jnp.broadcast_to(b_h2, (16,))
    logits_pad = _edge_head(a_tab, t_tab, comb, w2b, b2b)
    return logits_pad[:E]


# speedup vs baseline: 1.0411x; 1.0411x over previous
import functools

import jax
import jax.numpy as jnp
from jax import lax
from jax.experimental import pallas as pl
from jax.experimental.pallas import tpu as pltpu
from jax.experimental.pallas import tpu_sc as plsc

NA = 50000
NT = 50000
E = 800000
H = 64

RW = 32
CW = 16
NP = 50008
ACC_R = 50048
TPT = ACC_R // 16
E_PAD = 802816
EROWS = E_PAD // 128
TILE_EROWS = EROWS // 16
EH_TILE_EROWS = EROWS // 32

FKB = 2
FCHUNK = FKB * 128
NFCH = TILE_EROWS // FKB

CKB = 8
CCHUNK = CKB * 128
NCCH = TILE_EROWS // CKB

EKB = 2
ECHUNK = EKB * 128
NECH = EH_TILE_EROWS // EKB

ENC_R = 2000
POST_R = 2000

_SC_PARAMS = pltpu.CompilerParams(use_tc_tiling_on_sc=False,
                                  needs_layout_passes=False)


def _enc_body(x_ref, w_ref, b_ref, g_ref, be_ref, h_ref, gtab_ref):
    x = x_ref[...]
    y = jnp.dot(x, w_ref[...].T, preferred_element_type=jnp.float32)
    y = y + b_ref[...]
    m = jnp.mean(y, axis=-1, keepdims=True)
    v = jnp.mean((y - m) ** 2, axis=-1, keepdims=True)
    y = (y - m) * lax.rsqrt(v + 1e-5) * g_ref[...] + be_ref[...]
    h = jnp.maximum(y, 0.0)
    h_ref[...] = h
    gtab_ref[...] = jnp.stack([h[:, :RW], h[:, RW:]], axis=0)


def _encode(x, w, b, g, be, n):
    return pl.pallas_call(
        _enc_body,
        grid=(n // ENC_R,),
        in_specs=[
            pl.BlockSpec((ENC_R, 128), lambda i: (i, 0)),
            pl.BlockSpec((H, 128), lambda i: (0, 0)),
            pl.BlockSpec((H,), lambda i: (0,)),
            pl.BlockSpec((H,), lambda i: (0,)),
            pl.BlockSpec((H,), lambda i: (0,)),
        ],
        out_specs=[
            pl.BlockSpec((ENC_R, H), lambda i: (i, 0)),
            pl.BlockSpec((2, ENC_R, RW), lambda i: (0, i, 0)),
        ],
        out_shape=[
            jax.ShapeDtypeStruct((n, H), jnp.float32),
            jax.ShapeDtypeStruct((2, n, RW), jnp.float32),
        ],
    )(x, w, b, g, be)


def _ring_driver(n, S, G):
    for k in range(3):
        G(k, k, first=True)

    def body(jj, _):
        k0 = 3 * jj
        for u in range(3):
            S(k0 + u, u)
            G(k0 + u + 3, u)
        return _

    nloop = (n - 4) // 3
    lax.fori_loop(0, nloop, body, None)
    for k in range(3 * nloop, n):
        S(k, k % 3)
        kg = k + 3
        if kg < n:
            G(kg, kg % 3)


def _feat_phase(c, s, table, gpar, out, acc, cidx, rows, gsems, ssems,
                zf_hbm, comb):
    pltpu.sync_copy(zf_hbm, acc.at[pl.ds(s * TPT, TPT)])
    plsc.subcore_barrier()
    base = 2 * s * TILE_EROWS
    cnp = c * NP

    def G(k, b, first=False):
        if not first:
            for j in range(FKB):
                pltpu.make_async_copy(rows[b].at[pl.ds(j * 128, 128)],
                                      acc.at[pl.ds(0, 128)], ssems[b]).wait()
        r0 = base + k * (2 * FKB)
        pltpu.sync_copy(comb.at[pl.ds(r0, 2 * FKB)], cidx[b])
        for j in range(FKB):
            row = 2 * j + gpar
            for q in range(8):
                sl = cidx[b][row, pl.ds(q * 16, 16)]
                cidx[b][row, pl.ds(q * 16, 16)] = sl + cnp
            pltpu.async_copy(table.at[cidx[b].at[row]],
                             rows[b].at[pl.ds(j * 128, 128)], gsems[b])

    def S(k, b):
        for j in range(FKB):
            pltpu.make_async_copy(table.at[pl.ds(0, 128)],
                                  rows[b].at[pl.ds(j * 128, 128)],
                                  gsems[b]).wait()
        for j in range(FKB):
            pltpu.async_copy(rows[b].at[pl.ds(j * 128, 128)],
                             acc.at[cidx[b].at[2 * j + 1 - gpar]], ssems[b],
                             add=True)

    _ring_driver(NFCH, S, G)
    for b in range(3):
        for j in range(FKB):
            pltpu.make_async_copy(rows[b].at[pl.ds(j * 128, 128)],
                                  acc.at[pl.ds(0, 128)], ssems[b]).wait()
    plsc.subcore_barrier()
    pltpu.sync_copy(acc.at[pl.ds(s * TPT, TPT)],
                    out.at[c, pl.ds(s * TPT, TPT)])
    plsc.subcore_barrier()


def _cnt_body(comb, ones_hbm, zc_hbm, cnt_out, acc,
              cidx0, cidx1, cidx2, ones_v, ss0, ss1, ss2):
    c = lax.axis_index("c")
    s = lax.axis_index("s")
    cidx = (cidx0, cidx1, cidx2)
    ssems = (ss0, ss1, ss2)
    pltpu.sync_copy(zc_hbm, acc.at[pl.ds(s * TPT, TPT)])
    pltpu.sync_copy(ones_hbm, ones_v)
    plsc.subcore_barrier()
    base = 2 * s * TILE_EROWS

    def G(k, b, first=False):
        if not first:
            for j in range(FKB):
                pltpu.make_async_copy(ones_v, acc.at[pl.ds(0, 128)],
                                      ssems[b]).wait()
        r0 = base + k * (2 * FKB)
        pltpu.sync_copy(comb.at[pl.ds(r0, 2 * FKB)], cidx[b])

    def S(k, b):
        @pl.when(c == 0)
        def _():
            for j in range(FKB):
                pltpu.async_copy(ones_v, acc.at[cidx[b].at[2 * j + 1]],
                                 ssems[b], add=True)

        @pl.when(c == 1)
        def _():
            for j in range(FKB):
                pltpu.async_copy(ones_v, acc.at[cidx[b].at[2 * j]],
                                 ssems[b], add=True)

    _ring_driver(NFCH, S, G)
    for b in range(3):
        for j in range(FKB):
            pltpu.make_async_copy(ones_v, acc.at[pl.ds(0, 128)],
                                  ssems[b]).wait()
    plsc.subcore_barrier()
    pltpu.sync_copy(acc.at[pl.ds(s * TPT, TPT)],
                    cnt_out.at[c, pl.ds(s * TPT, TPT)])


def _counts(comb, ones_hbm, zc_hbm):
    mesh = plsc.VectorSubcoreMesh(core_axis_name="c", subcore_axis_name="s")
    f = functools.partial(
        pl.kernel,
        out_type=jax.ShapeDtypeStruct((2, ACC_R, CW), jnp.float32),
        mesh=mesh,
        compiler_params=_SC_PARAMS,
        scratch_types=[
            pltpu.VMEM_SHARED((ACC_R, CW), jnp.float32),
            pltpu.VMEM((2 * FKB, 128), jnp.int32),
            pltpu.VMEM((2 * FKB, 128), jnp.int32),
            pltpu.VMEM((2 * FKB, 128), jnp.int32),
            pltpu.VMEM((128, CW), jnp.float32),
            pltpu.SemaphoreType.DMA,
            pltpu.SemaphoreType.DMA,
            pltpu.SemaphoreType.DMA,
        ],
    )(_cnt_body)
    return f(comb, ones_hbm, zc_hbm)


def _feat_body(g_a, g_t, comb, zf_hbm,
               s_t, s_a, acc, cidx0, cidx1, cidx2, rows0, rows1, rows2,
               gs0, gs1, gs2, ss0, ss1, ss2):
    c = lax.axis_index("c")
    s = lax.axis_index("s")
    cidx = (cidx0, cidx1, cidx2)
    rows = (rows0, rows1, rows2)
    gsems = (gs0, gs1, gs2)
    ssems = (ss0, ss1, ss2)
    _feat_phase(c, s, g_a, 0, s_t, acc, cidx, rows, gsems, ssems,
                zf_hbm, comb)
    _feat_phase(c, s, g_t, 1, s_a, acc, cidx, rows, gsems, ssems,
                zf_hbm, comb)


def _feat(g_a, g_t, comb, zf_hbm):
    mesh = plsc.VectorSubcoreMesh(core_axis_name="c", subcore_axis_name="s")
    f = functools.partial(
        pl.kernel,
        out_type=[
            jax.ShapeDtypeStruct((2, ACC_R, RW), jnp.float32),
            jax.ShapeDtypeStruct((2, ACC_R, RW), jnp.float32),
        ],
        mesh=mesh,
        compiler_params=_SC_PARAMS,
        scratch_types=[
            pltpu.VMEM_SHARED((ACC_R, RW), jnp.float32),
            pltpu.VMEM((2 * FKB, 128), jnp.int32),
            pltpu.VMEM((2 * FKB, 128), jnp.int32),
            pltpu.VMEM((2 * FKB, 128), jnp.int32),
            pltpu.VMEM((FCHUNK, RW), jnp.float32),
            pltpu.VMEM((FCHUNK, RW), jnp.float32),
            pltpu.VMEM((FCHUNK, RW), jnp.float32),
            pltpu.SemaphoreType.DMA,
            pltpu.SemaphoreType.DMA,
            pltpu.SemaphoreType.DMA,
            pltpu.SemaphoreType.DMA,
            pltpu.SemaphoreType.DMA,
            pltpu.SemaphoreType.DMA,
        ],
    )(_feat_body)
    return f(g_a, g_t, comb, zf_hbm)


def _post_body(st_ref, sa_ref, cnt_ref, ha_ref, ht_ref, wlst_ref, blst_ref,
               wrst_ref, wlts_ref, blts_ref, wrts_ref, wh1_ref, bh1_ref,
               a_ref, t_ref):
    st = st_ref[...]
    sa = sa_ref[...]
    cnt = cnt_ref[...]
    cnt_t = jnp.maximum(cnt[0, :, 0:1], 1.0)
    cnt_a = jnp.maximum(cnt[1, :, 0:1], 1.0)
    agg_t = jnp.concatenate([st[0], st[1]], axis=1) / cnt_t
    agg_a = jnp.concatenate([sa[0], sa[1]], axis=1) / cnt_a
    new_t = (jnp.dot(agg_t, wlst_ref[...].T, preferred_element_type=jnp.float32)
             + blst_ref[...]
             + jnp.dot(ht_ref[...], wrst_ref[...].T, preferred_element_type=jnp.float32))
    new_a = (jnp.dot(agg_a, wlts_ref[...].T, preferred_element_type=jnp.float32)
             + blts_ref[...]
             + jnp.dot(ha_ref[...], wrts_ref[...].T, preferred_element_type=jnp.float32))
    wh1 = wh1_ref[...]
    a_ref[...] = (jnp.dot(new_a, wh1[:, :H].T, preferred_element_type=jnp.float32)
                  + bh1_ref[...])
    t_ref[...] = jnp.dot(new_t, wh1[:, H:].T, preferred_element_type=jnp.float32)


def _post(s_t, s_a, cnt, h_a, h_t, wlst, blst, wrst, wlts, blts, wrts,
          wh1, bh1):
    return pl.pallas_call(
        _post_body,
        grid=(NA // POST_R,),
        in_specs=[
            pl.BlockSpec((2, POST_R, RW), lambda i: (0, i, 0)),
            pl.BlockSpec((2, POST_R, RW), lambda i: (0, i, 0)),
            pl.BlockSpec((2, POST_R, CW), lambda i: (0, i, 0)),
            pl.BlockSpec((POST_R, H), lambda i: (i, 0)),
            pl.BlockSpec((POST_R, H), lambda i: (i, 0)),
            pl.BlockSpec((H, H), lambda i: (0, 0)),
            pl.BlockSpec((H,), lambda i: (0,)),
            pl.BlockSpec((H, H), lambda i: (0, 0)),
            pl.BlockSpec((H, H), lambda i: (0, 0)),
            pl.BlockSpec((H,), lambda i: (0,)),
            pl.BlockSpec((H, H), lambda i: (0, 0)),
            pl.BlockSpec((H, 2 * H), lambda i: (0, 0)),
            pl.BlockSpec((H,), lambda i: (0,)),
        ],
        out_specs=[
            pl.BlockSpec((POST_R, H), lambda i: (i, 0)),
            pl.BlockSpec((POST_R, H), lambda i: (i, 0)),
        ],
        out_shape=[
            jax.ShapeDtypeStruct((ACC_R, H), jnp.float32),
            jax.ShapeDtypeStruct((ACC_R, H), jnp.float32),
        ],
    )(s_t, s_a, cnt, h_a, h_t, wlst, blst, wrst, wlts, blts, wrts, wh1, bh1)


def _edge_body(a_tab, t_tab, comb, w2_hbm, b2_hbm, logits,
               cidx0, cidx1, cidx2, ar0, ar1, ar2, tr0, tr1, tr2,
               out0, out1, out2, w2_v, b2_v, tile,
               sg0, sg1, sg2, sw0, sw1, sw2):
    c = lax.axis_index("c")
    s = lax.axis_index("s")
    wid = s * 2 + c
    base = 2 * wid * EH_TILE_EROWS
    obase = wid * (EH_TILE_EROWS * 128)
    cidx = (cidx0, cidx1, cidx2)
    ar = (ar0, ar1, ar2)
    tr = (tr0, tr1, tr2)
    outs = (out0, out1, out2)
    sg = (sg0, sg1, sg2)
    sw = (sw0, sw1, sw2)
    pltpu.sync_copy(w2_hbm, w2_v)
    pltpu.sync_copy(b2_hbm, b2_v)
    for b in range(3):
        pltpu.async_copy(outs[b], logits.at[pl.ds(E, ECHUNK)], sw[b])

    w2v = [w2_v[pl.ds(q * 16, 16)] for q in range(4)]
    b2vec = b2_v[...]
    iot = lax.iota(jnp.int32, 16)

    def G(k, b, first=False):
        r0 = base + k * (2 * EKB)
        pltpu.sync_copy(comb.at[pl.ds(r0, 2 * EKB)], cidx[b])
        for j in range(EKB):
            pltpu.async_copy(a_tab.at[cidx[b].at[2 * j]],
                             ar[b].at[pl.ds(j * 128, 128)], sg[b])
            pltpu.async_copy(t_tab.at[cidx[b].at[2 * j + 1]],
                             tr[b].at[pl.ds(j * 128, 128)], sg[b])

    def S(k, b):
        for j in range(EKB):
            pltpu.make_async_copy(a_tab.at[pl.ds(0, 128)],
                                  ar[b].at[pl.ds(j * 128, 128)], sg[b]).wait()
            pltpu.make_async_copy(t_tab.at[pl.ds(0, 128)],
                                  tr[b].at[pl.ds(j * 128, 128)], sg[b]).wait()
        out = outs[b]
        arows = ar[b]
        trows = tr[b]
        pltpu.make_async_copy(out, logits.at[pl.ds(E, ECHUNK)], sw[b]).wait()

        def group(g, _):
            e0 = g * 16
            for u in range(16):
                e = e0 + u
                t = None
                for q in range(4):
                    av = arows[e, pl.ds(q * 16, 16)]
                    tv = trows[e, pl.ds(q * 16, 16)]
                    r = jnp.maximum(av + tv, 0.0) * w2v[q]
                    t = r if t is None else t + r
                tile[u, :] = t
            res = b2vec
            for q in range(16):
                col = plsc.load_gather(tile, [iot, jnp.full((16,), q, jnp.int32)])
                res = res + col
            out[pl.ds(e0, 16)] = res
            return _

        lax.fori_loop(0, ECHUNK // 16, group, None)
        pltpu.async_copy(out, logits.at[pl.ds(obase + k * ECHUNK, ECHUNK)],
                         sw[b])

    _ring_driver(NECH, S, G)
    for b in range(3):
        pltpu.make_async_copy(outs[b], logits.at[pl.ds(E, ECHUNK)],
                              sw[b]).wait()


def _edge_head(a_tab, t_tab, comb, w2b, b2b):
    mesh = plsc.VectorSubcoreMesh(core_axis_name="c", subcore_axis_name="s")
    f = functools.partial(
        pl.kernel,
        out_type=jax.ShapeDtypeStruct((E_PAD,), jnp.float32),
        mesh=mesh,
        compiler_params=_SC_PARAMS,
        scratch_types=[
            pltpu.VMEM((2 * EKB, 128), jnp.int32),
            pltpu.VMEM((2 * EKB, 128), jnp.int32),
            pltpu.VMEM((2 * EKB, 128), jnp.int32),
            pltpu.VMEM((ECHUNK, H), jnp.float32),
            pltpu.VMEM((ECHUNK, H), jnp.float32),
            pltpu.VMEM((ECHUNK, H), jnp.float32),
            pltpu.VMEM((ECHUNK, H), jnp.float32),
            pltpu.VMEM((ECHUNK, H), jnp.float32),
            pltpu.VMEM((ECHUNK, H), jnp.float32),
            pltpu.VMEM((ECHUNK,), jnp.float32),
            pltpu.VMEM((ECHUNK,), jnp.float32),
            pltpu.VMEM((ECHUNK,), jnp.float32),
            pltpu.VMEM((H,), jnp.float32),
            pltpu.VMEM((16,), jnp.float32),
            pltpu.VMEM((16, 16), jnp.float32),
            pltpu.SemaphoreType.DMA,
            pltpu.SemaphoreType.DMA,
            pltpu.SemaphoreType.DMA,
            pltpu.SemaphoreType.DMA,
            pltpu.SemaphoreType.DMA,
            pltpu.SemaphoreType.DMA,
        ],
    )(_edge_body)
    return f(a_tab, t_tab, comb, w2b, b2b)


def kernel(x_agent, x_task, W_enc_a, b_enc_a, g_ln_a, be_ln_a, W_enc_t,
           b_enc_t, g_ln_t, be_ln_t, Wl_st, bl_st, Wr_st, Wl_ts, bl_ts,
           Wr_ts, W_h1, b_h1, W_h2, b_h2, edge_index):
    h_a, g_a = _encode(x_agent, W_enc_a, b_enc_a, g_ln_a, be_ln_a, NA)
    h_t, g_t = _encode(x_task, W_enc_t, b_enc_t, g_ln_t, be_ln_t, NT)
    zpad = jnp.zeros((2, NP - NA, RW), jnp.float32)
    g_a_flat = jnp.concatenate([g_a, zpad], axis=1).reshape(2 * NP, RW)
    g_t_flat = jnp.concatenate([g_t, zpad], axis=1).reshape(2 * NP, RW)

    src = edge_index[0]
    dst = edge_index[1]
    pad = E_PAD - E
    srcp = jnp.concatenate([src, jnp.full((pad,), NA, jnp.int32)])
    dstp = jnp.concatenate([dst, jnp.full((pad,), NA, jnp.int32)])
    comb = jnp.stack([srcp.reshape(EROWS, 128),
                      dstp.reshape(EROWS, 128)], axis=1)
    comb = comb.reshape(2 * EROWS, 128)

    ones_hbm = jnp.ones((128, CW), jnp.float32)
    zc_hbm = jnp.zeros((TPT, CW), jnp.float32)
    zf_hbm = jnp.zeros((TPT, RW), jnp.float32)

    cnt = _counts(comb, ones_hbm, zc_hbm)
    s_t, s_a = _feat(g_a_flat, g_t_flat, comb, zf_hbm)

    a_tab, t_tab = _post(s_t, s_a, cnt, h_a, h_t, Wl_st, bl_st, Wr_st,
                         Wl_ts, bl_ts, Wr_ts, W_h1, b_h1)

    w2b = W_h2[0]
    b2b = jnp.broadcast_to(b_h2, (16,))
    logits_pad = _edge_head(a_tab, t_tab, comb, w2b, b2b)
    return logits_pad[:E]


# speedup vs baseline: 1.1706x; 1.1244x over previous
import functools

import jax
import jax.numpy as jnp
from jax import lax
from jax.experimental import pallas as pl
from jax.experimental.pallas import tpu as pltpu
from jax.experimental.pallas import tpu_sc as plsc

NA = 50000
NT = 50000
E = 800000
H = 64

RW = 32
CW = 16
NP = 50008
ACC_R = 50048
TPT = ACC_R // 16
E_PAD = 802816
EROWS = E_PAD // 128
TILE_EROWS = EROWS // 16
EH_TILE_EROWS = EROWS // 32

FKB = 2
FCHUNK = FKB * 128
NFCH = TILE_EROWS // FKB

CKB = 8
CCHUNK = CKB * 128
NCCH = TILE_EROWS // CKB

EKB = 2
ECHUNK = EKB * 128
NECH = EH_TILE_EROWS // EKB

ENC_R = 2000
POST_R = 2000

_SC_PARAMS = pltpu.CompilerParams(use_tc_tiling_on_sc=False,
                                  needs_layout_passes=False)


def _enc_body(x_ref, w_ref, b_ref, g_ref, be_ref, h_ref, gtab_ref):
    x = x_ref[...]
    y = jnp.dot(x, w_ref[...].T, preferred_element_type=jnp.float32)
    y = y + b_ref[...]
    m = jnp.mean(y, axis=-1, keepdims=True)
    v = jnp.mean((y - m) ** 2, axis=-1, keepdims=True)
    y = (y - m) * lax.rsqrt(v + 1e-5) * g_ref[...] + be_ref[...]
    h = jnp.maximum(y, 0.0)
    h_ref[...] = h
    gtab_ref[...] = jnp.stack([h[:, :RW], h[:, RW:]], axis=0)


def _encode(x, w, b, g, be, n):
    return pl.pallas_call(
        _enc_body,
        grid=(n // ENC_R,),
        in_specs=[
            pl.BlockSpec((ENC_R, 128), lambda i: (i, 0)),
            pl.BlockSpec((H, 128), lambda i: (0, 0)),
            pl.BlockSpec((H,), lambda i: (0,)),
            pl.BlockSpec((H,), lambda i: (0,)),
            pl.BlockSpec((H,), lambda i: (0,)),
        ],
        out_specs=[
            pl.BlockSpec((ENC_R, H), lambda i: (i, 0)),
            pl.BlockSpec((2, ENC_R, RW), lambda i: (0, i, 0)),
        ],
        out_shape=[
            jax.ShapeDtypeStruct((n, H), jnp.float32),
            jax.ShapeDtypeStruct((2, n, RW), jnp.float32),
        ],
    )(x, w, b, g, be)


def _ring_driver(n, S, G):
    for k in range(3):
        G(k, k, first=True)

    def body(jj, _):
        k0 = 3 * jj
        for u in range(3):
            S(k0 + u, u)
            G(k0 + u + 3, u)
        return _

    nloop = (n - 4) // 3
    lax.fori_loop(0, nloop, body, None)
    for k in range(3 * nloop, n):
        S(k, k % 3)
        kg = k + 3
        if kg < n:
            G(kg, kg % 3)


def _feat_phase(c, s, table, gpar, out, acc, cidx, rows, sidx,
                gsems, ssems, isems, zf_hbm, comb):
    pltpu.sync_copy(zf_hbm, acc.at[pl.ds(s * TPT, TPT)])
    plsc.subcore_barrier()
    base = 2 * s * TILE_EROWS
    cnp = c * NP

    def stage(k, b):
        r0 = base + k * (2 * FKB)
        pltpu.async_copy(comb.at[pl.ds(r0, 2 * FKB)], cidx[b], isems[b])

    def G(k, b, first=False):
        if not first:
            for j in range(FKB):
                pltpu.make_async_copy(rows[b].at[pl.ds(j * 128, 128)],
                                      acc.at[pl.ds(0, 128)], ssems[b]).wait()
        else:
            stage(k, b)
        pltpu.make_async_copy(comb.at[pl.ds(0, 2 * FKB)], cidx[b],
                              isems[b]).wait()
        for j in range(FKB):
            row = 2 * j + gpar
            srow = 2 * j + 1 - gpar
            for q in range(8):
                sl = cidx[b][row, pl.ds(q * 16, 16)]
                cidx[b][row, pl.ds(q * 16, 16)] = sl + cnp
                sidx[b][j, pl.ds(q * 16, 16)] = cidx[b][srow, pl.ds(q * 16, 16)]
            pltpu.async_copy(table.at[cidx[b].at[row]],
                             rows[b].at[pl.ds(j * 128, 128)], gsems[b])

    def S(k, b):
        for j in range(FKB):
            pltpu.make_async_copy(table.at[pl.ds(0, 128)],
                                  rbuf_wait(b, j), gsems[b]).wait()
        if not (isinstance(k, int) and k + 3 >= NFCH):
            stage(k + 3, b)
        for j in range(FKB):
            pltpu.async_copy(rows[b].at[pl.ds(j * 128, 128)],
                             acc.at[sidx[b].at[j]], ssems[b], add=True)

    def rbuf_wait(b, j):
        return rows[b].at[pl.ds(j * 128, 128)]

    _ring_driver(NFCH, S, G)
    for b in range(3):
        for j in range(FKB):
            pltpu.make_async_copy(rows[b].at[pl.ds(j * 128, 128)],
                                  acc.at[pl.ds(0, 128)], ssems[b]).wait()
    plsc.subcore_barrier()
    pltpu.sync_copy(acc.at[pl.ds(s * TPT, TPT)],
                    out.at[c, pl.ds(s * TPT, TPT)])
    plsc.subcore_barrier()


def _cnt_body(comb, ones_hbm, zc_hbm, cnt_out, acc,
              cidx0, cidx1, cidx2, sx0, sx1, sx2, ones_v,
              ss0, ss1, ss2, is0, is1, is2):
    c = lax.axis_index("c")
    s = lax.axis_index("s")
    cidx = (cidx0, cidx1, cidx2)
    sidx = (sx0, sx1, sx2)
    ssems = (ss0, ss1, ss2)
    isems = (is0, is1, is2)
    pltpu.sync_copy(zc_hbm, acc.at[pl.ds(s * TPT, TPT)])
    pltpu.sync_copy(ones_hbm, ones_v)
    plsc.subcore_barrier()
    base = 2 * s * TILE_EROWS

    def stage(k, b):
        r0 = base + k * (2 * FKB)
        pltpu.async_copy(comb.at[pl.ds(r0, 2 * FKB)], cidx[b], isems[b])

    def G(k, b, first=False):
        if not first:
            for j in range(FKB):
                pltpu.make_async_copy(ones_v, acc.at[pl.ds(0, 128)],
                                      ssems[b]).wait()
        else:
            stage(k, b)
        pltpu.make_async_copy(comb.at[pl.ds(0, 2 * FKB)], cidx[b],
                              isems[b]).wait()
        for j in range(2 * FKB):
            for q in range(8):
                sidx[b][j, pl.ds(q * 16, 16)] = cidx[b][j, pl.ds(q * 16, 16)]

    def S(k, b):
        if not (isinstance(k, int) and k + 3 >= NFCH):
            stage(k + 3, b)

        @pl.when(c == 0)
        def _():
            for j in range(FKB):
                pltpu.async_copy(ones_v, acc.at[sidx[b].at[2 * j + 1]],
                                 ssems[b], add=True)

        @pl.when(c == 1)
        def _():
            for j in range(FKB):
                pltpu.async_copy(ones_v, acc.at[sidx[b].at[2 * j]],
                                 ssems[b], add=True)

    _ring_driver(NFCH, S, G)
    for b in range(3):
        for j in range(FKB):
            pltpu.make_async_copy(ones_v, acc.at[pl.ds(0, 128)],
                                  ssems[b]).wait()
    plsc.subcore_barrier()
    pltpu.sync_copy(acc.at[pl.ds(s * TPT, TPT)],
                    cnt_out.at[c, pl.ds(s * TPT, TPT)])


def _counts(comb, ones_hbm, zc_hbm):
    mesh = plsc.VectorSubcoreMesh(core_axis_name="c", subcore_axis_name="s")
    f = functools.partial(
        pl.kernel,
        out_type=jax.ShapeDtypeStruct((2, ACC_R, CW), jnp.float32),
        mesh=mesh,
        compiler_params=_SC_PARAMS,
        scratch_types=[
            pltpu.VMEM_SHARED((ACC_R, CW), jnp.float32),
            pltpu.VMEM((2 * FKB, 128), jnp.int32),
            pltpu.VMEM((2 * FKB, 128), jnp.int32),
            pltpu.VMEM((2 * FKB, 128), jnp.int32),
            pltpu.VMEM((2 * FKB, 128), jnp.int32),
            pltpu.VMEM((2 * FKB, 128), jnp.int32),
            pltpu.VMEM((2 * FKB, 128), jnp.int32),
            pltpu.VMEM((128, CW), jnp.float32),
            pltpu.SemaphoreType.DMA,
            pltpu.SemaphoreType.DMA,
            pltpu.SemaphoreType.DMA,
            pltpu.SemaphoreType.DMA,
            pltpu.SemaphoreType.DMA,
            pltpu.SemaphoreType.DMA,
        ],
    )(_cnt_body)
    return f(comb, ones_hbm, zc_hbm)


def _feat_body(g_a, g_t, comb, zf_hbm,
               s_t, s_a, acc, cidx0, cidx1, cidx2, rows0, rows1, rows2,
               sx0, sx1, sx2, gs0, gs1, gs2, ss0, ss1, ss2, is0, is1, is2):
    c = lax.axis_index("c")
    s = lax.axis_index("s")
    cidx = (cidx0, cidx1, cidx2)
    rows = (rows0, rows1, rows2)
    sidx = (sx0, sx1, sx2)
    gsems = (gs0, gs1, gs2)
    ssems = (ss0, ss1, ss2)
    isems = (is0, is1, is2)
    _feat_phase(c, s, g_a, 0, s_t, acc, cidx, rows, sidx, gsems, ssems,
                isems, zf_hbm, comb)
    _feat_phase(c, s, g_t, 1, s_a, acc, cidx, rows, sidx, gsems, ssems,
                isems, zf_hbm, comb)


def _feat(g_a, g_t, comb, zf_hbm):
    mesh = plsc.VectorSubcoreMesh(core_axis_name="c", subcore_axis_name="s")
    f = functools.partial(
        pl.kernel,
        out_type=[
            jax.ShapeDtypeStruct((2, ACC_R, RW), jnp.float32),
            jax.ShapeDtypeStruct((2, ACC_R, RW), jnp.float32),
        ],
        mesh=mesh,
        compiler_params=_SC_PARAMS,
        scratch_types=[
            pltpu.VMEM_SHARED((ACC_R, RW), jnp.float32),
            pltpu.VMEM((2 * FKB, 128), jnp.int32),
            pltpu.VMEM((2 * FKB, 128), jnp.int32),
            pltpu.VMEM((2 * FKB, 128), jnp.int32),
            pltpu.VMEM((FCHUNK, RW), jnp.float32),
            pltpu.VMEM((FCHUNK, RW), jnp.float32),
            pltpu.VMEM((FCHUNK, RW), jnp.float32),
            pltpu.VMEM((FKB, 128), jnp.int32),
            pltpu.VMEM((FKB, 128), jnp.int32),
            pltpu.VMEM((FKB, 128), jnp.int32),
            pltpu.SemaphoreType.DMA,
            pltpu.SemaphoreType.DMA,
            pltpu.SemaphoreType.DMA,
            pltpu.SemaphoreType.DMA,
            pltpu.SemaphoreType.DMA,
            pltpu.SemaphoreType.DMA,
            pltpu.SemaphoreType.DMA,
            pltpu.SemaphoreType.DMA,
            pltpu.SemaphoreType.DMA,
        ],
    )(_feat_body)
    return f(g_a, g_t, comb, zf_hbm)


def _post_body(st_ref, sa_ref, cnt_ref, ha_ref, ht_ref, wlst_ref, blst_ref,
               wrst_ref, wlts_ref, blts_ref, wrts_ref, wh1_ref, bh1_ref,
               a_ref, t_ref):
    st = st_ref[...]
    sa = sa_ref[...]
    cnt = cnt_ref[...]
    cnt_t = jnp.maximum(cnt[0, :, 0:1], 1.0)
    cnt_a = jnp.maximum(cnt[1, :, 0:1], 1.0)
    agg_t = jnp.concatenate([st[0], st[1]], axis=1) / cnt_t
    agg_a = jnp.concatenate([sa[0], sa[1]], axis=1) / cnt_a
    new_t = (jnp.dot(agg_t, wlst_ref[...].T, preferred_element_type=jnp.float32)
             + blst_ref[...]
             + jnp.dot(ht_ref[...], wrst_ref[...].T, preferred_element_type=jnp.float32))
    new_a = (jnp.dot(agg_a, wlts_ref[...].T, preferred_element_type=jnp.float32)
             + blts_ref[...]
             + jnp.dot(ha_ref[...], wrts_ref[...].T, preferred_element_type=jnp.float32))
    wh1 = wh1_ref[...]
    a_ref[...] = (jnp.dot(new_a, wh1[:, :H].T, preferred_element_type=jnp.float32)
                  + bh1_ref[...])
    t_ref[...] = jnp.dot(new_t, wh1[:, H:].T, preferred_element_type=jnp.float32)


def _post(s_t, s_a, cnt, h_a, h_t, wlst, blst, wrst, wlts, blts, wrts,
          wh1, bh1):
    return pl.pallas_call(
        _post_body,
        grid=(NA // POST_R,),
        in_specs=[
            pl.BlockSpec((2, POST_R, RW), lambda i: (0, i, 0)),
            pl.BlockSpec((2, POST_R, RW), lambda i: (0, i, 0)),
            pl.BlockSpec((2, POST_R, CW), lambda i: (0, i, 0)),
            pl.BlockSpec((POST_R, H), lambda i: (i, 0)),
            pl.BlockSpec((POST_R, H), lambda i: (i, 0)),
            pl.BlockSpec((H, H), lambda i: (0, 0)),
            pl.BlockSpec((H,), lambda i: (0,)),
            pl.BlockSpec((H, H), lambda i: (0, 0)),
            pl.BlockSpec((H, H), lambda i: (0, 0)),
            pl.BlockSpec((H,), lambda i: (0,)),
            pl.BlockSpec((H, H), lambda i: (0, 0)),
            pl.BlockSpec((H, 2 * H), lambda i: (0, 0)),
            pl.BlockSpec((H,), lambda i: (0,)),
        ],
        out_specs=[
            pl.BlockSpec((POST_R, H), lambda i: (i, 0)),
            pl.BlockSpec((POST_R, H), lambda i: (i, 0)),
        ],
        out_shape=[
            jax.ShapeDtypeStruct((ACC_R, H), jnp.float32),
            jax.ShapeDtypeStruct((ACC_R, H), jnp.float32),
        ],
    )(s_t, s_a, cnt, h_a, h_t, wlst, blst, wrst, wlts, blts, wrts, wh1, bh1)


def _edge_body(a_tab, t_tab, comb, w2_hbm, b2_hbm, logits,
               cidx0, cidx1, cidx2, ar0, ar1, ar2, tr0, tr1, tr2,
               out0, out1, out2, w2_v, b2_v, tile,
               sg0, sg1, sg2, sw0, sw1, sw2, si0, si1, si2):
    c = lax.axis_index("c")
    s = lax.axis_index("s")
    wid = s * 2 + c
    base = 2 * wid * EH_TILE_EROWS
    obase = wid * (EH_TILE_EROWS * 128)
    cidx = (cidx0, cidx1, cidx2)
    ar = (ar0, ar1, ar2)
    tr = (tr0, tr1, tr2)
    outs = (out0, out1, out2)
    sg = (sg0, sg1, sg2)
    sw = (sw0, sw1, sw2)
    si = (si0, si1, si2)
    pltpu.sync_copy(w2_hbm, w2_v)
    pltpu.sync_copy(b2_hbm, b2_v)
    for b in range(3):
        pltpu.async_copy(outs[b], logits.at[pl.ds(E, ECHUNK)], sw[b])

    w2v = [w2_v[pl.ds(q * 16, 16)] for q in range(4)]
    b2vec = b2_v[...]
    iot = lax.iota(jnp.int32, 16)

    def stage(k, b):
        r0 = base + k * (2 * EKB)
        pltpu.async_copy(comb.at[pl.ds(r0, 2 * EKB)], cidx[b], si[b])

    def G(k, b, first=False):
        if first:
            stage(k, b)
        pltpu.make_async_copy(comb.at[pl.ds(0, 2 * EKB)], cidx[b],
                              si[b]).wait()
        for j in range(EKB):
            pltpu.async_copy(a_tab.at[cidx[b].at[2 * j]],
                             ar[b].at[pl.ds(j * 128, 128)], sg[b])
            pltpu.async_copy(t_tab.at[cidx[b].at[2 * j + 1]],
                             tr[b].at[pl.ds(j * 128, 128)], sg[b])

    def S(k, b):
        for j in range(EKB):
            pltpu.make_async_copy(a_tab.at[pl.ds(0, 128)],
                                  ar[b].at[pl.ds(j * 128, 128)], sg[b]).wait()
            pltpu.make_async_copy(t_tab.at[pl.ds(0, 128)],
                                  tr[b].at[pl.ds(j * 128, 128)], sg[b]).wait()
        if not (isinstance(k, int) and k + 3 >= NECH):
            stage(k + 3, b)
        out = outs[b]
        arows = ar[b]
        trows = tr[b]
        pltpu.make_async_copy(out, logits.at[pl.ds(E, ECHUNK)], sw[b]).wait()

        def group(g, _):
            e0 = g * 16
            for u in range(16):
                e = e0 + u
                t = None
                for q in range(4):
                    av = arows[e, pl.ds(q * 16, 16)]
                    tv = trows[e, pl.ds(q * 16, 16)]
                    r = jnp.maximum(av + tv, 0.0) * w2v[q]
                    t = r if t is None else t + r
                tile[u, :] = t
            res = b2vec
            for q in range(16):
                col = plsc.load_gather(tile, [iot, jnp.full((16,), q, jnp.int32)])
                res = res + col
            out[pl.ds(e0, 16)] = res
            return _

        lax.fori_loop(0, ECHUNK // 16, group, None)
        pltpu.async_copy(out, logits.at[pl.ds(obase + k * ECHUNK, ECHUNK)],
                         sw[b])

    _ring_driver(NECH, S, G)
    for b in range(3):
        pltpu.make_async_copy(outs[b], logits.at[pl.ds(E, ECHUNK)],
                              sw[b]).wait()


def _edge_head(a_tab, t_tab, comb, w2b, b2b):
    mesh = plsc.VectorSubcoreMesh(core_axis_name="c", subcore_axis_name="s")
    f = functools.partial(
        pl.kernel,
        out_type=jax.ShapeDtypeStruct((E_PAD,), jnp.float32),
        mesh=mesh,
        compiler_params=_SC_PARAMS,
        scratch_types=[
            pltpu.VMEM((2 * EKB, 128), jnp.int32),
            pltpu.VMEM((2 * EKB, 128), jnp.int32),
            pltpu.VMEM((2 * EKB, 128), jnp.int32),
            pltpu.VMEM((ECHUNK, H), jnp.float32),
            pltpu.VMEM((ECHUNK, H), jnp.float32),
            pltpu.VMEM((ECHUNK, H), jnp.float32),
            pltpu.VMEM((ECHUNK, H), jnp.float32),
            pltpu.VMEM((ECHUNK, H), jnp.float32),
            pltpu.VMEM((ECHUNK, H), jnp.float32),
            pltpu.VMEM((ECHUNK,), jnp.float32),
            pltpu.VMEM((ECHUNK,), jnp.float32),
            pltpu.VMEM((ECHUNK,), jnp.float32),
            pltpu.VMEM((H,), jnp.float32),
            pltpu.VMEM((16,), jnp.float32),
            pltpu.VMEM((16, 16), jnp.float32),
            pltpu.SemaphoreType.DMA,
            pltpu.SemaphoreType.DMA,
            pltpu.SemaphoreType.DMA,
            pltpu.SemaphoreType.DMA,
            pltpu.SemaphoreType.DMA,
            pltpu.SemaphoreType.DMA,
            pltpu.SemaphoreType.DMA,
            pltpu.SemaphoreType.DMA,
            pltpu.SemaphoreType.DMA,
        ],
    )(_edge_body)
    return f(a_tab, t_tab, comb, w2b, b2b)


def kernel(x_agent, x_task, W_enc_a, b_enc_a, g_ln_a, be_ln_a, W_enc_t,
           b_enc_t, g_ln_t, be_ln_t, Wl_st, bl_st, Wr_st, Wl_ts, bl_ts,
           Wr_ts, W_h1, b_h1, W_h2, b_h2, edge_index):
    h_a, g_a = _encode(x_agent, W_enc_a, b_enc_a, g_ln_a, be_ln_a, NA)
    h_t, g_t = _encode(x_task, W_enc_t, b_enc_t, g_ln_t, be_ln_t, NT)
    zpad = jnp.zeros((2, NP - NA, RW), jnp.float32)
    g_a_flat = jnp.concatenate([g_a, zpad], axis=1).reshape(2 * NP, RW)
    g_t_flat = jnp.concatenate([g_t, zpad], axis=1).reshape(2 * NP, RW)

    src = edge_index[0]
    dst = edge_index[1]
    pad = E_PAD - E
    srcp = jnp.concatenate([src, jnp.full((pad,), NA, jnp.int32)])
    dstp = jnp.concatenate([dst, jnp.full((pad,), NA, jnp.int32)])
    comb = jnp.stack([srcp.reshape(EROWS, 128),
                      dstp.reshape(EROWS, 128)], axis=1)
    comb = comb.reshape(2 * EROWS, 128)

    ones_hbm = jnp.ones((128, CW), jnp.float32)
    zc_hbm = jnp.zeros((TPT, CW), jnp.float32)
    zf_hbm = jnp.zeros((TPT, RW), jnp.float32)

    cnt = _counts(comb, ones_hbm, zc_hbm)
    s_t, s_a = _feat(g_a_flat, g_t_flat, comb, zf_hbm)

    a_tab, t_tab = _post(s_t, s_a, cnt, h_a, h_t, Wl_st, bl_st, Wr_st,
                         Wl_ts, bl_ts, Wr_ts, W_h1, b_h1)

    w2b = W_h2[0]
    b2b = jnp.broadcast_to(b_h2, (16,))
    logits_pad = _edge_head(a_tab, t_tab, comb, w2b, b2b)
    return logits_pad[:E]
